# baseline (speedup 1.0000x reference)
import functools

import jax
import jax.numpy as jnp
from jax import lax
from jax.experimental import pallas as pl
from jax.experimental.pallas import tpu as pltpu
from jax.experimental.pallas import tpu_sc as plsc

_NC = 2
_NS = 16
_NW = _NC * _NS
_LANES = 16


def _sc_embed_sum(idx_t, table_flat, bias16, batch, num_fields):
    b_per_w = batch // _NW
    n_idx = num_fields * b_per_w
    n_vec = b_per_w // _LANES
    mesh = plsc.VectorSubcoreMesh(core_axis_name="c", subcore_axis_name="s")

    @functools.partial(
        pl.kernel,
        out_type=jax.ShapeDtypeStruct((batch,), jnp.float32),
        mesh=mesh,
        scratch_types=[
            pltpu.VMEM((n_idx,), jnp.int32),
            pltpu.VMEM((n_idx,), jnp.float32),
            pltpu.VMEM((b_per_w,), jnp.float32),
            pltpu.VMEM((_LANES,), jnp.float32),
            pltpu.SemaphoreType.DMA,
        ],
    )
    def k(idx_hbm, table_hbm, bias_hbm, out_hbm, idx_v, vals_v, out_v, bias_v, sem):
        wid = lax.axis_index("s") * _NC + lax.axis_index("c")
        base = wid * b_per_w
        pltpu.sync_copy(bias_hbm, bias_v)
        pltpu.sync_copy(idx_hbm.at[wid], idx_v)
        pltpu.async_copy(table_hbm.at[idx_v], vals_v, sem).wait()

        def body(v, carry):
            acc = bias_v[...]
            for f in range(num_fields):
                acc = acc + vals_v[pl.ds(f * b_per_w + v * _LANES, _LANES)]
            out_v[pl.ds(v * _LANES, _LANES)] = acc
            return carry

        lax.fori_loop(0, n_vec, body, 0)
        pltpu.sync_copy(out_v, out_hbm.at[pl.ds(base, b_per_w)])

    return k(idx_t, table_flat, bias16)


def kernel(x, fc_weight, bias):
    batch, num_fields = x.shape
    b_per_w = batch // _NW
    idx_t = (
        x.reshape(_NW, b_per_w, num_fields)
        .transpose(0, 2, 1)
        .reshape(_NW, num_fields * b_per_w)
    )
    bias16 = jnp.broadcast_to(bias, (_LANES,))
    table_flat = fc_weight[:, 0]
    out = _sc_embed_sum(idx_t, table_flat, bias16, batch, num_fields)
    return out.reshape(batch, fc_weight.shape[1])


# speedup vs baseline: 1.0007x; 1.0007x over previous
import functools

import jax
import jax.numpy as jnp
from jax import lax
from jax.experimental import pallas as pl
from jax.experimental.pallas import tpu as pltpu
from jax.experimental.pallas import tpu_sc as plsc

_NC = 2
_NS = 16
_NW = _NC * _NS
_LANES = 16


def _sc_embed_sum(idx_t, table_flat, bias16, batch, num_fields):
    b_per_w = batch // _NW
    n_idx = num_fields * b_per_w
    n_vec = b_per_w // _LANES
    mesh = plsc.VectorSubcoreMesh(core_axis_name="c", subcore_axis_name="s")

    @functools.partial(
        pl.kernel,
        out_type=jax.ShapeDtypeStruct((batch,), jnp.float32),
        mesh=mesh,
        scratch_types=[
            pltpu.VMEM((n_idx,), jnp.int32),
            pltpu.VMEM((n_idx,), jnp.float32),
            pltpu.VMEM((b_per_w,), jnp.float32),
            pltpu.VMEM((_LANES,), jnp.float32),
            pltpu.SemaphoreType.DMA,
        ],
    )
    def k(idx_hbm, table_hbm, bias_hbm, out_hbm, idx_v, vals_v, out_v, bias_v, sem):
        wid = lax.axis_index("s") * _NC + lax.axis_index("c")
        base = wid * b_per_w
        pltpu.sync_copy(bias_hbm, bias_v)
        pltpu.sync_copy(idx_hbm.at[wid], idx_v)
        pltpu.async_copy(table_hbm.at[idx_v], vals_v, sem).wait()

        def body(v, carry):
            acc = bias_v[...]
            for f in range(num_fields):
                acc = acc + vals_v[pl.ds(f * b_per_w + v * _LANES, _LANES)]
            out_v[pl.ds(v * _LANES, _LANES)] = acc
            return carry

        lax.fori_loop(0, n_vec, body, 0)
        pltpu.sync_copy(out_v, out_hbm.at[pl.ds(base, b_per_w)])

    return k(idx_t, table_flat, bias16)


def kernel(x, fc_weight, bias):
    batch, num_fields = x.shape
    b_per_w = batch // _NW
    idx_t = (
        x.reshape(_NW, b_per_w, num_fields)
        .transpose(0, 2, 1)
        .reshape(_NW, num_fields * b_per_w)
    )
    table_flat = fc_weight.T.reshape(-1)
    bias16 = jnp.broadcast_to(bias, (_LANES,))
    out = _sc_embed_sum(idx_t, table_flat, bias16, batch, num_fields)
    return out.reshape(batch, fc_weight.shape[1])


# speedup vs baseline: 1.0011x; 1.0004x over previous
import functools

import jax
import jax.numpy as jnp
from jax import lax
from jax.experimental import pallas as pl
from jax.experimental.pallas import tpu as pltpu
from jax.experimental.pallas import tpu_sc as plsc

_NC = 2
_NS = 16
_NW = _NC * _NS
_LANES = 16


def _sc_embed_sum(idx_t, table_flat, bias16, batch, num_fields):
    b_per_w = batch // _NW
    n_idx = num_fields * b_per_w
    n_vec = b_per_w // _LANES
    mesh = plsc.VectorSubcoreMesh(core_axis_name="c", subcore_axis_name="s")

    @functools.partial(
        pl.kernel,
        out_type=jax.ShapeDtypeStruct((batch,), jnp.float32),
        mesh=mesh,
        compiler_params=pltpu.CompilerParams(use_tc_tiling_on_sc=False),
        scratch_types=[
            pltpu.VMEM((n_idx,), jnp.int32),
            pltpu.VMEM((n_idx,), jnp.float32),
            pltpu.VMEM((b_per_w,), jnp.float32),
            pltpu.VMEM((_LANES,), jnp.float32),
            pltpu.SemaphoreType.DMA,
        ],
    )
    def k(idx_hbm, table_hbm, bias_hbm, out_hbm, idx_v, vals_v, out_v, bias_v, sem):
        wid = lax.axis_index("s") * _NC + lax.axis_index("c")
        base = wid * b_per_w
        pltpu.sync_copy(bias_hbm, bias_v)
        pltpu.sync_copy(idx_hbm.at[wid], idx_v)
        pltpu.async_copy(table_hbm.at[idx_v], vals_v, sem).wait()

        def body(v, carry):
            acc = bias_v[...]
            for f in range(num_fields):
                acc = acc + vals_v[pl.ds(f * b_per_w + v * _LANES, _LANES)]
            out_v[pl.ds(v * _LANES, _LANES)] = acc
            return carry

        lax.fori_loop(0, n_vec, body, 0)
        pltpu.sync_copy(out_v, out_hbm.at[pl.ds(base, b_per_w)])

    return k(idx_t, table_flat, bias16)


def kernel(x, fc_weight, bias):
    batch, num_fields = x.shape
    b_per_w = batch // _NW
    idx_t = (
        x.reshape(_NW, b_per_w, num_fields)
        .transpose(0, 2, 1)
        .reshape(_NW, num_fields * b_per_w)
    )
    table_flat = fc_weight.T.reshape(-1)
    bias16 = jnp.broadcast_to(bias, (_LANES,))
    out = _sc_embed_sum(idx_t, table_flat, bias16, batch, num_fields)
    return out.reshape(batch, 1)


# speedup vs baseline: 1.0051x; 1.0040x over previous
import functools

import jax
import jax.numpy as jnp
from jax import lax
from jax.experimental import pallas as pl
from jax.experimental.pallas import tpu as pltpu
from jax.experimental.pallas import tpu_sc as plsc

_NC = 2
_NS = 16
_NW = _NC * _NS
_LANES = 16
_F0 = 13


def _sc_embed_sum(idx_t, table_flat, bias16, batch, num_fields):
    b_per_w = batch // _NW
    n_idx = num_fields * b_per_w
    n_vec = b_per_w // _LANES
    h0 = _F0 * b_per_w
    h1 = n_idx - h0
    mesh = plsc.VectorSubcoreMesh(core_axis_name="c", subcore_axis_name="s")

    @functools.partial(
        pl.kernel,
        out_type=jax.ShapeDtypeStruct((batch,), jnp.float32),
        mesh=mesh,
        compiler_params=pltpu.CompilerParams(use_tc_tiling_on_sc=False),
        scratch_types=[
            pltpu.VMEM((n_idx,), jnp.int32),
            pltpu.VMEM((n_idx,), jnp.float32),
            pltpu.VMEM((b_per_w,), jnp.float32),
            pltpu.VMEM((_LANES,), jnp.float32),
            pltpu.SemaphoreType.DMA,
            pltpu.SemaphoreType.DMA,
            pltpu.SemaphoreType.DMA,
            pltpu.SemaphoreType.DMA,
        ],
    )
    def k(idx_hbm, table_hbm, bias_hbm, out_hbm, idx_v, vals_v, out_v, bias_v,
          si0, si1, sg0, sg1):
        wid = lax.axis_index("s") * _NC + lax.axis_index("c")
        base = wid * b_per_w

        ci0 = pltpu.async_copy(
            idx_hbm.at[wid, pl.ds(0, h0)], idx_v.at[pl.ds(0, h0)], si0)
        ci1 = pltpu.async_copy(
            idx_hbm.at[wid, pl.ds(h0, h1)], idx_v.at[pl.ds(h0, h1)], si1)
        pltpu.sync_copy(bias_hbm, bias_v)
        ci0.wait()
        g0 = pltpu.async_copy(
            table_hbm.at[idx_v.at[pl.ds(0, h0)]], vals_v.at[pl.ds(0, h0)], sg0)
        ci1.wait()
        g1 = pltpu.async_copy(
            table_hbm.at[idx_v.at[pl.ds(h0, h1)]], vals_v.at[pl.ds(h0, h1)],
            sg1)
        g0.wait()

        def body0(v, carry):
            acc = bias_v[...]
            for f in range(_F0):
                acc = acc + vals_v[pl.ds(f * b_per_w + v * _LANES, _LANES)]
            out_v[pl.ds(v * _LANES, _LANES)] = acc
            return carry

        lax.fori_loop(0, n_vec, body0, 0)
        g1.wait()

        def body1(v, carry):
            acc = out_v[pl.ds(v * _LANES, _LANES)]
            for f in range(_F0, num_fields):
                acc = acc + vals_v[pl.ds(f * b_per_w + v * _LANES, _LANES)]
            out_v[pl.ds(v * _LANES, _LANES)] = acc
            return carry

        lax.fori_loop(0, n_vec, body1, 0)
        pltpu.sync_copy(out_v, out_hbm.at[pl.ds(base, b_per_w)])

    return k(idx_t, table_flat, bias16)


def kernel(x, fc_weight, bias):
    batch, num_fields = x.shape
    b_per_w = batch // _NW
    idx_t = (
        x.reshape(_NW, b_per_w, num_fields)
        .transpose(0, 2, 1)
        .reshape(_NW, num_fields * b_per_w)
    )
    bias16 = jnp.broadcast_to(bias, (_LANES,))
    out = _sc_embed_sum(idx_t, fc_weight.reshape(-1), bias16, batch, num_fields)
    return out.reshape(batch, 1)


# speedup vs baseline: 1.0306x; 1.0254x over previous
import functools

import jax
import jax.numpy as jnp
from jax import lax
from jax.experimental import pallas as pl
from jax.experimental.pallas import tpu as pltpu
from jax.experimental.pallas import tpu_sc as plsc

_NC = 2
_NS = 16
_NW = _NC * _NS
_LANES = 16
_F0 = 13


def _sc_embed_sum(idx_t, table2d, bias16, batch, num_fields):
    b_per_w = batch // _NW
    n_idx = num_fields * b_per_w
    n_vec = b_per_w // _LANES
    h0 = _F0 * b_per_w
    h1 = n_idx - h0
    mesh = plsc.VectorSubcoreMesh(core_axis_name="c", subcore_axis_name="s")

    @functools.partial(
        pl.kernel,
        out_type=jax.ShapeDtypeStruct((batch,), jnp.float32),
        mesh=mesh,
        compiler_params=pltpu.CompilerParams(use_tc_tiling_on_sc=False),
        scratch_types=[
            pltpu.VMEM((n_idx,), jnp.int32),
            pltpu.VMEM((n_idx,), jnp.float32),
            pltpu.VMEM((b_per_w,), jnp.float32),
            pltpu.VMEM((_LANES,), jnp.float32),
            pltpu.SemaphoreType.DMA,
            pltpu.SemaphoreType.DMA,
            pltpu.SemaphoreType.DMA,
            pltpu.SemaphoreType.DMA,
        ],
    )
    def k(idx_hbm, table_hbm, bias_hbm, out_hbm, idx_v, vals_v, out_v, bias_v,
          si0, si1, sg0, sg1):
        wid = lax.axis_index("s") * _NC + lax.axis_index("c")
        base = wid * b_per_w

        ci0 = pltpu.async_copy(
            idx_hbm.at[wid, pl.ds(0, h0)], idx_v.at[pl.ds(0, h0)], si0)
        ci1 = pltpu.async_copy(
            idx_hbm.at[wid, pl.ds(h0, h1)], idx_v.at[pl.ds(h0, h1)], si1)
        pltpu.sync_copy(bias_hbm, bias_v)
        ci0.wait()
        tbl = table_hbm.at[0]
        g0 = pltpu.async_copy(
            tbl.at[idx_v.at[pl.ds(0, h0)]], vals_v.at[pl.ds(0, h0)], sg0)
        ci1.wait()
        g1 = pltpu.async_copy(
            tbl.at[idx_v.at[pl.ds(h0, h1)]], vals_v.at[pl.ds(h0, h1)],
            sg1)
        g0.wait()

        def body0(v, carry):
            acc = bias_v[...]
            for f in range(_F0):
                acc = acc + vals_v[pl.ds(f * b_per_w + v * _LANES, _LANES)]
            out_v[pl.ds(v * _LANES, _LANES)] = acc
            return carry

        lax.fori_loop(0, n_vec, body0, 0)
        g1.wait()

        def body1(v, carry):
            acc = out_v[pl.ds(v * _LANES, _LANES)]
            for f in range(_F0, num_fields):
                acc = acc + vals_v[pl.ds(f * b_per_w + v * _LANES, _LANES)]
            out_v[pl.ds(v * _LANES, _LANES)] = acc
            return carry

        lax.fori_loop(0, n_vec, body1, 0)
        pltpu.sync_copy(out_v, out_hbm.at[pl.ds(base, b_per_w)])

    return k(idx_t, table2d, bias16)


def kernel(x, fc_weight, bias):
    batch, num_fields = x.shape
    b_per_w = batch // _NW
    idx_t = (
        x.reshape(_NW, b_per_w, num_fields)
        .transpose(0, 2, 1)
        .reshape(_NW, num_fields * b_per_w)
    )
    bias16 = jnp.broadcast_to(bias, (_LANES,))
    out = _sc_embed_sum(idx_t, fc_weight.reshape(1, -1), bias16, batch, num_fields)
    return out.reshape(batch, 1)
